# baseline (speedup 1.0000x reference)
import functools

import jax
import jax.numpy as jnp
from jax import lax
from jax.experimental import pallas as pl
from jax.experimental.pallas import tpu as pltpu
from jax.experimental.pallas import tpu_sc as plsc

_B, _S, _D = 4, 8192, 1024
_L = 16


def _sc_kernel(emb_hbm, lm_hbm, out_hbm, mask_v, rows_v, sem):
    cid = lax.axis_index("c")
    sid = lax.axis_index("s")

    @pl.when(jnp.logical_and(cid == 0, sid < _B))
    def _():
        b = sid
        pltpu.sync_copy(lm_hbm.at[b], mask_v)

        def body(i, acc):
            return acc + mask_v[pl.ds(i * _L, _L)]

        acc = lax.fori_loop(0, _S // _L, body, jnp.zeros((_L,), jnp.int32))
        total = acc[0]
        for i in range(1, _L):
            total = total + acc[i]
        idx = b * _S + total - 1
        idx_vec = jnp.full((_L,), idx, dtype=jnp.int32)
        pltpu.async_copy(emb_hbm.at[idx_vec], rows_v, sem).wait()
        pltpu.sync_copy(rows_v.at[0], out_hbm.at[b])


def kernel(embeddings, labels, embeddings_mask, labels_mask):
    del labels, embeddings_mask
    lm = labels_mask.astype(jnp.int32)
    emb2 = embeddings.reshape(_B * _S, _D)

    mesh = plsc.VectorSubcoreMesh(core_axis_name="c", subcore_axis_name="s")
    run = functools.partial(
        pl.kernel,
        mesh=mesh,
        out_type=jax.ShapeDtypeStruct((_B, _D), jnp.float32),
        scratch_types=[
            pltpu.VMEM((_S,), jnp.int32),
            pltpu.VMEM((_L, _D), jnp.float32),
            pltpu.SemaphoreType.DMA,
        ],
    )(_sc_kernel)
    return run(emb2, lm)
